# baseline (speedup 1.0000x reference)
import functools

import jax
import jax.numpy as jnp
import numpy as np
from jax import lax
from jax.experimental import pallas as pl
from jax.experimental.pallas import tpu as pltpu
from jax.experimental.pallas import tpu_sc as plsc

MAX_COUNT = 255
EMBED_DIM = 128
NUM_CORES = 2
NUM_SUBCORES = 16
NUM_WORKERS = NUM_CORES * NUM_SUBCORES
LANES = 16
CHUNK = 128
NBUF = 2
TABLE_ROWS = MAX_COUNT + 1


@functools.cache
def _build(n_chunks: int, fields: int):
    b_per_w = n_chunks * CHUNK
    total = NUM_WORKERS * b_per_w
    n_rounds = n_chunks // NBUF
    band = 8 * fields
    period = int(np.lcm(band, CHUNK))
    chunks_per_period = period // CHUNK
    assert n_chunks % chunks_per_period == 0 and b_per_w % band == 0
    mesh = plsc.VectorSubcoreMesh(core_axis_name="c", subcore_axis_name="s")

    @functools.partial(
        pl.kernel,
        out_type=jax.ShapeDtypeStruct((total, EMBED_DIM), jnp.float32),
        mesh=mesh,
        compiler_params=pltpu.CompilerParams(needs_layout_passes=False),
        scratch_types=[
            pltpu.VMEM_SHARED(
                (NUM_SUBCORES * TABLE_ROWS, EMBED_DIM), jnp.float32
            ),
            pltpu.VMEM((b_per_w,), jnp.int32),
            pltpu.VMEM((period,), jnp.int32),
            [pltpu.VMEM((CHUNK,), jnp.int32) for _ in range(NBUF)],
            [pltpu.VMEM((CHUNK, EMBED_DIM), jnp.float32) for _ in range(NBUF)],
            [pltpu.SemaphoreType.DMA for _ in range(NBUF)],
            [pltpu.SemaphoreType.DMA for _ in range(NBUF)],
        ],
    )
    def gather_kernel(
        table_hbm, idx_hbm, rel_hbm, out_hbm,
        table_sp, idx_v, rel_v, pidx, rows, semg, sems,
    ):
        sid = lax.axis_index("s")
        wid = sid * NUM_CORES + lax.axis_index("c")
        base = wid * b_per_w

        tbase = sid * TABLE_ROWS
        pltpu.sync_copy(table_hbm, table_sp.at[pl.ds(tbase, TABLE_ROWS)])

        pltpu.sync_copy(idx_hbm.at[wid], idx_v)
        pltpu.sync_copy(rel_hbm, rel_v)

        def build_pidx(b, chunk):
            group = chunk // chunks_per_period
            rem = chunk % chunks_per_period
            goff = group * period
            for q16 in range(CHUNK // LANES):
                sv = rel_v[pl.ds(rem * CHUNK + q16 * LANES, LANES)] + goff
                vals = plsc.load_gather(idx_v, [sv]) + tbase
                pidx[b][pl.ds(q16 * LANES, LANES)] = vals

        def start_gather(b):
            pltpu.async_copy(table_sp.at[pidx[b]], rows[b], semg[b])

        def wait_gather(b):
            pltpu.make_async_copy(
                out_hbm.at[pl.ds(0, CHUNK)], rows[b], semg[b]
            ).wait()

        def start_store(b, chunk):
            pltpu.async_copy(
                rows[b], out_hbm.at[pl.ds(base + chunk * CHUNK, CHUNK)], sems[b]
            )

        def wait_store(b):
            pltpu.make_async_copy(
                rows[b], out_hbm.at[pl.ds(0, CHUNK)], sems[b]
            ).wait()

        for b in range(NBUF):
            build_pidx(b, b)
            start_gather(b)

        def round_body(j, carry):
            for b in range(NBUF):
                wait_gather(b)
                start_store(b, j * NBUF + b)
            for b in range(NBUF):
                wait_store(b)
                build_pidx(b, (j + 1) * NBUF + b)
                start_gather(b)
            return carry

        lax.fori_loop(0, n_rounds - 1, round_body, 0)

        for b in range(NBUF):
            wait_gather(b)
            start_store(b, (n_rounds - 1) * NBUF + b)
        for b in range(NBUF):
            wait_store(b)

    return gather_kernel


def kernel(tensor, int_to_feat_matrix, extra_embeddings):
    batch, fields = tensor.shape
    total = batch * fields
    b_per_w = total // NUM_WORKERS
    band = 8 * fields
    period = int(np.lcm(band, CHUNK))
    q = np.arange(period)
    r = q % band
    rel = (q // band) * band + (r % 8) * fields + r // 8
    rel = jnp.asarray(rel, dtype=jnp.int32)
    table = jnp.concatenate([int_to_feat_matrix, extra_embeddings], axis=0)
    idx = tensor.astype(jnp.int32).reshape(NUM_WORKERS, b_per_w)
    out = _build(b_per_w // CHUNK, fields)(table, idx, rel)
    bands = batch // 8
    return (
        out.reshape(bands, fields, 8, EMBED_DIM)
        .transpose(0, 2, 1, 3)
        .reshape(batch, fields * EMBED_DIM)
    )


# speedup vs baseline: 1.0445x; 1.0445x over previous
import functools

import jax
import jax.numpy as jnp
import numpy as np
from jax import lax
from jax.experimental import pallas as pl
from jax.experimental.pallas import tpu as pltpu
from jax.experimental.pallas import tpu_sc as plsc

MAX_COUNT = 255
EMBED_DIM = 128
NUM_CORES = 2
NUM_SUBCORES = 16
NUM_WORKERS = NUM_CORES * NUM_SUBCORES
LANES = 16
CHUNK = 128
NBUF = 2
TABLE_ROWS = MAX_COUNT + 1


@functools.cache
def _build(n_chunks: int, fields: int):
    b_per_w = n_chunks * CHUNK
    total = NUM_WORKERS * b_per_w
    n_rounds = n_chunks // NBUF
    band = 8 * fields
    period = int(np.lcm(band, CHUNK))
    chunks_per_period = period // CHUNK
    assert n_chunks % chunks_per_period == 0 and b_per_w % band == 0
    mesh = plsc.VectorSubcoreMesh(core_axis_name="c", subcore_axis_name="s")

    @functools.partial(
        pl.kernel,
        out_type=jax.ShapeDtypeStruct((total, EMBED_DIM), jnp.float32),
        mesh=mesh,
        compiler_params=pltpu.CompilerParams(needs_layout_passes=False),
        scratch_types=[
            pltpu.VMEM_SHARED((TABLE_ROWS, EMBED_DIM), jnp.float32),
            pltpu.VMEM((b_per_w,), jnp.int32),
            pltpu.VMEM((period,), jnp.int32),
            [pltpu.VMEM((CHUNK,), jnp.int32) for _ in range(NBUF)],
            [pltpu.VMEM((CHUNK, EMBED_DIM), jnp.float32) for _ in range(NBUF)],
            [pltpu.SemaphoreType.DMA for _ in range(NBUF)],
            [pltpu.SemaphoreType.DMA for _ in range(NBUF)],
        ],
    )
    def gather_kernel(
        feat_hbm, extra_hbm, idx_hbm, rel_hbm, out_hbm,
        table_sp, idx_v, rel_v, pidx, rows, semg, sems,
    ):
        sid = lax.axis_index("s")
        wid = sid * NUM_CORES + lax.axis_index("c")
        base = wid * b_per_w

        @pl.when(sid == 0)
        def _():
            pltpu.sync_copy(feat_hbm, table_sp.at[pl.ds(0, MAX_COUNT)])
            pltpu.sync_copy(extra_hbm, table_sp.at[pl.ds(MAX_COUNT, 1)])

        pltpu.sync_copy(idx_hbm.at[wid], idx_v)
        pltpu.sync_copy(rel_hbm, rel_v)
        plsc.subcore_barrier()

        def build_pidx(b, chunk):
            group = chunk // chunks_per_period
            rem = chunk % chunks_per_period
            goff = group * period
            for q16 in range(CHUNK // LANES):
                sv = rel_v[pl.ds(rem * CHUNK + q16 * LANES, LANES)] + goff
                vals = plsc.load_gather(idx_v, [sv])
                pidx[b][pl.ds(q16 * LANES, LANES)] = vals

        def start_gather(b):
            pltpu.async_copy(table_sp.at[pidx[b]], rows[b], semg[b])

        def wait_gather(b):
            pltpu.make_async_copy(
                out_hbm.at[pl.ds(0, CHUNK)], rows[b], semg[b]
            ).wait()

        def start_store(b, chunk):
            pltpu.async_copy(
                rows[b], out_hbm.at[pl.ds(base + chunk * CHUNK, CHUNK)], sems[b]
            )

        def wait_store(b):
            pltpu.make_async_copy(
                rows[b], out_hbm.at[pl.ds(0, CHUNK)], sems[b]
            ).wait()

        for b in range(NBUF):
            build_pidx(b, b)
            start_gather(b)

        def round_body(j, carry):
            for b in range(NBUF):
                wait_gather(b)
                start_store(b, j * NBUF + b)
            for b in range(NBUF):
                wait_store(b)
                build_pidx(b, (j + 1) * NBUF + b)
                start_gather(b)
            return carry

        lax.fori_loop(0, n_rounds - 1, round_body, 0)

        for b in range(NBUF):
            wait_gather(b)
            start_store(b, (n_rounds - 1) * NBUF + b)
        for b in range(NBUF):
            wait_store(b)

    return gather_kernel


def kernel(tensor, int_to_feat_matrix, extra_embeddings):
    batch, fields = tensor.shape
    total = batch * fields
    b_per_w = total // NUM_WORKERS
    band = 8 * fields
    period = int(np.lcm(band, CHUNK))
    q = np.arange(period)
    r = q % band
    rel = (q // band) * band + (r % 8) * fields + r // 8
    rel = jnp.asarray(rel, dtype=jnp.int32)
    idx = tensor.astype(jnp.int32).reshape(NUM_WORKERS, b_per_w)
    out = _build(b_per_w // CHUNK, fields)(
        int_to_feat_matrix, extra_embeddings, idx, rel
    )
    bands = batch // 8
    return (
        out.reshape(bands, fields, 8, EMBED_DIM)
        .transpose(0, 2, 1, 3)
        .reshape(batch, fields * EMBED_DIM)
    )


# speedup vs baseline: 1.2494x; 1.1962x over previous
import functools

import jax
import jax.numpy as jnp
import numpy as np
from jax import lax
from jax.experimental import pallas as pl
from jax.experimental.pallas import tpu as pltpu
from jax.experimental.pallas import tpu_sc as plsc

MAX_COUNT = 255
EMBED_DIM = 128
NUM_CORES = 2
NUM_SUBCORES = 16
NUM_WORKERS = NUM_CORES * NUM_SUBCORES
LANES = 16
CHUNK = 64
NBUF = 8
TABLE_ROWS = MAX_COUNT + 1


@functools.cache
def _build(n_chunks: int, fields: int):
    b_per_w = n_chunks * CHUNK
    total = NUM_WORKERS * b_per_w
    n_rounds = n_chunks // NBUF
    band = 8 * fields
    period = int(np.lcm(band, CHUNK))
    chunks_per_period = period // CHUNK
    assert n_chunks % chunks_per_period == 0 and b_per_w % band == 0
    mesh = plsc.VectorSubcoreMesh(core_axis_name="c", subcore_axis_name="s")

    @functools.partial(
        pl.kernel,
        out_type=jax.ShapeDtypeStruct((total, EMBED_DIM), jnp.float32),
        mesh=mesh,
        compiler_params=pltpu.CompilerParams(needs_layout_passes=False),
        scratch_types=[
            pltpu.VMEM_SHARED((TABLE_ROWS, EMBED_DIM), jnp.float32),
            pltpu.VMEM((b_per_w,), jnp.int32),
            pltpu.VMEM((period,), jnp.int32),
            [pltpu.VMEM((CHUNK,), jnp.int32) for _ in range(NBUF)],
            [pltpu.VMEM((CHUNK, EMBED_DIM), jnp.float32) for _ in range(NBUF)],
            [pltpu.SemaphoreType.DMA for _ in range(NBUF)],
            [pltpu.SemaphoreType.DMA for _ in range(NBUF)],
        ],
    )
    def gather_kernel(
        feat_hbm, extra_hbm, idx_hbm, rel_hbm, out_hbm,
        table_sp, idx_v, rel_v, pidx, rows, semg, sems,
    ):
        sid = lax.axis_index("s")
        wid = sid * NUM_CORES + lax.axis_index("c")
        base = wid * b_per_w

        @pl.when(sid == 0)
        def _():
            pltpu.sync_copy(feat_hbm, table_sp.at[pl.ds(0, MAX_COUNT)])
            pltpu.sync_copy(extra_hbm, table_sp.at[pl.ds(MAX_COUNT, 1)])

        pltpu.sync_copy(idx_hbm.at[wid], idx_v)
        pltpu.sync_copy(rel_hbm, rel_v)
        plsc.subcore_barrier()

        def build_pidx(b, chunk):
            group = chunk // chunks_per_period
            rem = chunk % chunks_per_period
            goff = group * period
            for q16 in range(CHUNK // LANES):
                sv = rel_v[pl.ds(rem * CHUNK + q16 * LANES, LANES)] + goff
                vals = plsc.load_gather(idx_v, [sv])
                pidx[b][pl.ds(q16 * LANES, LANES)] = vals

        def start_gather(b):
            pltpu.async_copy(table_sp.at[pidx[b]], rows[b], semg[b])

        def wait_gather(b):
            pltpu.make_async_copy(
                out_hbm.at[pl.ds(0, CHUNK)], rows[b], semg[b]
            ).wait()

        def start_store(b, chunk):
            pltpu.async_copy(
                rows[b], out_hbm.at[pl.ds(base + chunk * CHUNK, CHUNK)], sems[b]
            )

        def wait_store(b):
            pltpu.make_async_copy(
                rows[b], out_hbm.at[pl.ds(0, CHUNK)], sems[b]
            ).wait()

        for b in range(NBUF):
            build_pidx(b, b)
            start_gather(b)

        def round_body(j, carry):
            for b in range(NBUF):
                wait_gather(b)
                start_store(b, j * NBUF + b)
            for b in range(NBUF):
                wait_store(b)
                build_pidx(b, (j + 1) * NBUF + b)
                start_gather(b)
            return carry

        lax.fori_loop(0, n_rounds - 1, round_body, 0)

        for b in range(NBUF):
            wait_gather(b)
            start_store(b, (n_rounds - 1) * NBUF + b)
        for b in range(NBUF):
            wait_store(b)

    return gather_kernel


def kernel(tensor, int_to_feat_matrix, extra_embeddings):
    batch, fields = tensor.shape
    total = batch * fields
    b_per_w = total // NUM_WORKERS
    band = 8 * fields
    period = int(np.lcm(band, CHUNK))
    q = np.arange(period)
    r = q % band
    rel = (q // band) * band + (r % 8) * fields + r // 8
    rel = jnp.asarray(rel, dtype=jnp.int32)
    idx = tensor.astype(jnp.int32).reshape(NUM_WORKERS, b_per_w)
    out = _build(b_per_w // CHUNK, fields)(
        int_to_feat_matrix, extra_embeddings, idx, rel
    )
    bands = batch // 8
    return (
        out.reshape(bands, fields, 8, EMBED_DIM)
        .transpose(0, 2, 1, 3)
        .reshape(batch, fields * EMBED_DIM)
    )


# speedup vs baseline: 1.2660x; 1.0133x over previous
import functools

import jax
import jax.numpy as jnp
import numpy as np
from jax import lax
from jax.experimental import pallas as pl
from jax.experimental.pallas import tpu as pltpu
from jax.experimental.pallas import tpu_sc as plsc

MAX_COUNT = 255
EMBED_DIM = 128
NUM_CORES = 2
NUM_SUBCORES = 16
NUM_WORKERS = NUM_CORES * NUM_SUBCORES
LANES = 16
CHUNK = 64
NBUF = 5
TABLE_ROWS = MAX_COUNT + 1


@functools.cache
def _build(n_chunks: int, fields: int):
    b_per_w = n_chunks * CHUNK
    total = NUM_WORKERS * b_per_w
    n_rounds = n_chunks // NBUF
    band = 8 * fields
    period = int(np.lcm(band, CHUNK))
    chunks_per_period = period // CHUNK
    assert n_chunks % chunks_per_period == 0 and b_per_w % band == 0
    mesh = plsc.VectorSubcoreMesh(core_axis_name="c", subcore_axis_name="s")

    @functools.partial(
        pl.kernel,
        out_type=jax.ShapeDtypeStruct((total, EMBED_DIM), jnp.float32),
        mesh=mesh,
        compiler_params=pltpu.CompilerParams(needs_layout_passes=False),
        scratch_types=[
            pltpu.VMEM_SHARED((TABLE_ROWS, EMBED_DIM), jnp.float32),
            pltpu.VMEM((b_per_w // 100, 100), jnp.int32),
            pltpu.VMEM((period,), jnp.int32),
            pltpu.VMEM((period,), jnp.int32),
            [pltpu.VMEM((CHUNK,), jnp.int32) for _ in range(NBUF)],
            [pltpu.VMEM((CHUNK, EMBED_DIM), jnp.float32) for _ in range(NBUF)],
            [pltpu.SemaphoreType.DMA for _ in range(NBUF)],
            [pltpu.SemaphoreType.DMA for _ in range(NBUF)],
        ],
    )
    def gather_kernel(
        feat_hbm, extra_hbm, idx_hbm, rrel_hbm, crel_hbm, out_hbm,
        table_sp, idx_v, rrel_v, crel_v, pidx, rows, semg, sems,
    ):
        sid = lax.axis_index("s")
        wid = sid * NUM_CORES + lax.axis_index("c")
        base = wid * b_per_w

        @pl.when(sid == 0)
        def _():
            pltpu.sync_copy(feat_hbm, table_sp.at[pl.ds(0, MAX_COUNT)])
            pltpu.sync_copy(extra_hbm, table_sp.at[pl.ds(MAX_COUNT, 1)])

        rows_per_w = b_per_w // 100
        pltpu.sync_copy(idx_hbm.at[pl.ds(wid * rows_per_w, rows_per_w)], idx_v)
        pltpu.sync_copy(rrel_hbm, rrel_v)
        pltpu.sync_copy(crel_hbm, crel_v)
        plsc.subcore_barrier()

        def build_pidx(b, chunk):
            group = chunk // chunks_per_period
            rem = chunk % chunks_per_period
            goff = group * (period // fields)
            for q16 in range(CHUNK // LANES):
                rv = rrel_v[pl.ds(rem * CHUNK + q16 * LANES, LANES)] + goff
                cv = crel_v[pl.ds(rem * CHUNK + q16 * LANES, LANES)]
                vals = plsc.load_gather(idx_v, [rv, cv])
                pidx[b][pl.ds(q16 * LANES, LANES)] = vals

        def start_gather(b):
            pltpu.async_copy(table_sp.at[pidx[b]], rows[b], semg[b])

        def wait_gather(b):
            pltpu.make_async_copy(
                out_hbm.at[pl.ds(0, CHUNK)], rows[b], semg[b]
            ).wait()

        def start_store(b, chunk):
            pltpu.async_copy(
                rows[b], out_hbm.at[pl.ds(base + chunk * CHUNK, CHUNK)], sems[b]
            )

        def wait_store(b):
            pltpu.make_async_copy(
                rows[b], out_hbm.at[pl.ds(0, CHUNK)], sems[b]
            ).wait()

        for b in range(NBUF):
            build_pidx(b, b)
            start_gather(b)

        def round_body(j, carry):
            for b in range(NBUF):
                wait_gather(b)
                start_store(b, j * NBUF + b)
            for b in range(NBUF):
                wait_store(b)
                build_pidx(b, (j + 1) * NBUF + b)
                start_gather(b)
            return carry

        lax.fori_loop(0, n_rounds - 1, round_body, 0)

        for b in range(NBUF):
            wait_gather(b)
            start_store(b, (n_rounds - 1) * NBUF + b)
        for b in range(NBUF):
            wait_store(b)

    return gather_kernel


def kernel(tensor, int_to_feat_matrix, extra_embeddings):
    batch, fields = tensor.shape
    total = batch * fields
    b_per_w = total // NUM_WORKERS
    band = 8 * fields
    period = int(np.lcm(band, CHUNK))
    q = np.arange(period)
    r = q % band
    rrel = jnp.asarray((q // band) * 8 + r % 8, dtype=jnp.int32)
    crel = jnp.asarray(r // 8, dtype=jnp.int32)
    idx = tensor.astype(jnp.int32)
    out = _build(b_per_w // CHUNK, fields)(
        int_to_feat_matrix, extra_embeddings, idx, rrel, crel
    )
    bands = batch // 8
    return (
        out.reshape(bands, fields, 8, EMBED_DIM)
        .transpose(0, 2, 1, 3)
        .reshape(batch, fields * EMBED_DIM)
    )
